# baseline (speedup 1.0000x reference)
import jax
import jax.numpy as jnp
from jax import lax
from jax.experimental import pallas as pl
from jax.experimental.pallas import tpu as pltpu
from jax.experimental.pallas import tpu_sc as plsc

K = 7
B = 4096
C = 1000
BB = 256

NC = 2
NS = 16
SC_ROWS = 512
TC_ROWS = B - SC_ROWS
CORE_ROWS = SC_ROWS // NC


def _majority_kernel(x_ref, out_ref):
    x = x_ref[...]
    am = jnp.argmax(x, axis=1).astype(jnp.int32)
    rows = [am[i] for i in range(K)]
    keys = []
    for j in range(K):
        cnt = (rows[0] == rows[j]).astype(jnp.int32)
        for i in range(1, K):
            cnt = cnt + (rows[i] == rows[j]).astype(jnp.int32)
        keys.append(cnt * 1024 - rows[j])
    best = keys[0]
    for j in range(1, K):
        best = jnp.maximum(best, keys[j])
    idx = jnp.where(keys[0] == best, 0, -1)
    for j in range(1, K):
        idx = jnp.maximum(idx, jnp.where(keys[j] == best, j, -1))
    out_ref[...] = idx.astype(jnp.int32)


def _sc_vote(rows):
    ones = jnp.full((16,), 1, jnp.int32)
    zeros = jnp.full((16,), 0, jnp.int32)
    neg1 = jnp.full((16,), -1, jnp.int32)
    k1024 = jnp.full((16,), 1024, jnp.int32)
    keys = []
    for j in range(K):
        cnt = jnp.where(rows[0] == rows[j], ones, zeros)
        for i in range(1, K):
            cnt = cnt + jnp.where(rows[i] == rows[j], ones, zeros)
        keys.append(cnt * k1024 - rows[j])
    best = keys[0]
    for j in range(1, K):
        best = jnp.maximum(best, keys[j])
    idx = jnp.where(keys[0] == best, zeros, neg1)
    for j in range(1, K):
        jvec = jnp.full((16,), j, jnp.int32)
        idx = jnp.maximum(idx, jnp.where(keys[j] == best, jvec, neg1))
    return idx


def _sc_majority_kernel(st_hbm, out_hbm, votes_sp, slab, votes_l, am_l,
                        out_l, sem):
    cid = lax.axis_index("c")
    sid = lax.axis_index("s")
    k = sid % K
    half = sid // K
    base = TC_ROWS + cid * CORE_ROWS + half * 128

    @pl.when(sid < 2 * K)
    def _argmax_stage():
        pltpu.async_copy(
            st_hbm.at[k, :, pl.ds(base, 128)], slab, sem
        ).wait()
        minf = jnp.full((16,), -jnp.inf, jnp.float32)
        zeros = jnp.full((16,), 0, jnp.int32)
        for g in range(8):
            def body(i, carry):
                m, am = carry
                for u in range(4):
                    cc = i * 4 + u
                    v = slab[cc, pl.ds(g * 16, 16)]
                    upd = v > m
                    cvec = jnp.full((16,), cc, jnp.int32)
                    am = jnp.where(upd, cvec, am)
                    m = jnp.where(upd, v, m)
                return m, am
            _, am_g = lax.fori_loop(0, C // 4, body, (minf, zeros))
            am_l[pl.ds(g * 16, 16)] = am_g
        pltpu.sync_copy(am_l, votes_sp.at[k, pl.ds(half * 128, 128)])

    plsc.subcore_barrier()

    pltpu.sync_copy(votes_sp, votes_l)
    rows = [votes_l[j, pl.ds(sid * 16, 16)] for j in range(K)]
    out_l[...] = _sc_vote(rows)
    pltpu.sync_copy(
        out_l, out_hbm.at[pl.ds(cid * CORE_ROWS + sid * 16, 16)]
    )


def kernel(scores):
    st = jnp.transpose(scores, (0, 2, 1))
    tc_idx = pl.pallas_call(
        _majority_kernel,
        grid=(TC_ROWS // BB,),
        in_specs=[pl.BlockSpec((K, C, BB), lambda i: (0, 0, i))],
        out_specs=pl.BlockSpec((BB,), lambda i: (i,)),
        out_shape=jax.ShapeDtypeStruct((TC_ROWS,), jnp.int32),
    )(st)
    sc_call = pl.kernel(
        _sc_majority_kernel,
        out_type=jax.ShapeDtypeStruct((SC_ROWS,), jnp.int32),
        mesh=plsc.VectorSubcoreMesh(core_axis_name="c", subcore_axis_name="s"),
        scratch_types=[
            pltpu.VMEM_SHARED((K, CORE_ROWS), jnp.int32),
            pltpu.VMEM((C, 128), jnp.float32),
            pltpu.VMEM((K, CORE_ROWS), jnp.int32),
            pltpu.VMEM((128,), jnp.int32),
            pltpu.VMEM((16,), jnp.int32),
            pltpu.SemaphoreType.DMA,
        ],
    )
    sc_idx = sc_call(st)
    return jnp.concatenate([tc_idx, sc_idx])


# speedup vs baseline: 1.4903x; 1.4903x over previous
import jax
import jax.numpy as jnp
from jax.experimental import pallas as pl

K = 7
B = 4096
C = 1000
BB = 256

GRID = (B // BB,)
IN_SPECS = [pl.BlockSpec((K, C, BB), lambda i: (0, 0, i))]
OUT_SPECS = pl.BlockSpec((BB,), lambda i: (i,))
OUT_SHAPE = jax.ShapeDtypeStruct((B,), jnp.int32)


def _majority_kernel(x_ref, out_ref):
    x = x_ref[...]
    am = jnp.argmax(x, axis=1).astype(jnp.int32)
    rows = [am[i] for i in range(K)]
    keys = []
    for j in range(K):
        cnt = (rows[0] == rows[j]).astype(jnp.int32)
        for i in range(1, K):
            cnt = cnt + (rows[i] == rows[j]).astype(jnp.int32)
        keys.append(cnt * 1024 - rows[j])
    best = keys[0]
    for j in range(1, K):
        best = jnp.maximum(best, keys[j])
    idx = jnp.where(keys[0] == best, 0, -1)
    for j in range(1, K):
        idx = jnp.maximum(idx, jnp.where(keys[j] == best, j, -1))
    out_ref[...] = idx.astype(jnp.int32)


def kernel(scores):
    st = jnp.transpose(scores, (0, 2, 1))
    out = pl.pallas_call(
        _majority_kernel,
        grid=GRID,
        in_specs=IN_SPECS,
        out_specs=OUT_SPECS,
        out_shape=OUT_SHAPE,
    )(st)
    return out
